# speedup vs baseline: 1.7545x; 1.7545x over previous
import functools

import jax
import jax.numpy as jnp
from jax.experimental import pallas as pl
from jax.experimental.pallas import tpu as pltpu

_BM = 512


def _gate_body(x_ref, w1_ref, b1_ref, w2_ref, b2_ref, sparse_ref, idx_ref):
    x = x_ref[...]
    h = jnp.dot(x, w1_ref[...], preferred_element_type=jnp.float32)
    h = h + b1_ref[...]
    h = h * jax.nn.sigmoid(h)
    logits = jnp.dot(h, w2_ref[...], preferred_element_type=jnp.float32)
    logits = logits + b2_ref[...]
    m = jnp.max(logits, axis=1, keepdims=True)
    e = jnp.exp(logits - m)
    probs = e / jnp.sum(e, axis=1, keepdims=True)
    lanes = jax.lax.broadcasted_iota(jnp.int32, probs.shape, 1)
    m1 = jnp.max(probs, axis=1, keepdims=True)
    i1 = jnp.min(jnp.where(probs == m1, lanes, probs.shape[1]), axis=1, keepdims=True)
    mask1 = lanes == i1
    rest = jnp.where(mask1, -jnp.inf, probs)
    m2 = jnp.max(rest, axis=1, keepdims=True)
    i2 = jnp.min(jnp.where(rest == m2, lanes, probs.shape[1]), axis=1, keepdims=True)
    mask2 = lanes == i2
    sparse_ref[...] = jnp.where(mask1 | mask2, probs, 0.0)
    idx_ref[...] = jnp.concatenate([i1, i2], axis=1)


@jax.jit
def kernel(x, W1, b1, W2, b2):
    n_tokens, hidden = x.shape
    n_mod = W2.shape[1]
    grid = (n_tokens // _BM,)
    sparse, idx = pl.pallas_call(
        _gate_body,
        grid=grid,
        in_specs=[
            pl.BlockSpec((_BM, hidden), lambda i: (i, 0)),
            pl.BlockSpec((hidden, W1.shape[1]), lambda i: (0, 0)),
            pl.BlockSpec((W1.shape[1],), lambda i: (0,)),
            pl.BlockSpec((W1.shape[1], n_mod), lambda i: (0, 0)),
            pl.BlockSpec((n_mod,), lambda i: (0,)),
        ],
        out_specs=[
            pl.BlockSpec((_BM, n_mod), lambda i: (i, 0)),
            pl.BlockSpec((_BM, 2), lambda i: (i, 0)),
        ],
        out_shape=[
            jax.ShapeDtypeStruct((n_tokens, n_mod), jnp.float32),
            jax.ShapeDtypeStruct((n_tokens, 2), jnp.int32),
        ],
    )(x, W1, b1, W2, b2)
    return (sparse, idx)
